# baseline (speedup 1.0000x reference)
import jax
import jax.numpy as jnp
from jax import lax
from jax.experimental import pallas as pl
from jax.experimental.pallas import tpu as pltpu
from jax.experimental.pallas import tpu_sc as plsc

N = 10000
NPAD = 10240
C = 128
E = 320000
NC = 2
NS = 16
L = 16
BLK = 128
NBUF = 2
CHUNK = 10
NB = 160
NCH = NB // CHUNK
EPT = NB * BLK
EPAD = EPT * NS
ROWS_PT = NPAD // NS
COPY_ROWS = 64


def _layer_body(table, comb_hbm, w_hbm, out, acc, cbuf0, cbuf1,
                wbuf0, wbuf1, rows0, rows1,
                gs0, gs1, ss0, ss1, cs0, cs1):
    c = lax.axis_index("c")
    s = lax.axis_index("s")
    rows = (rows0, rows1)
    gsem = (gs0, gs1)
    ssem = (ss0, ss1)
    cbuf = (cbuf0, cbuf1)
    wbuf = (wbuf0, wbuf1)
    csem = (cs0, cs1)
    t = c * NS + s
    r0 = s * ROWS_PT

    zv = jnp.zeros((L,), jnp.float32)

    @pl.loop(0, COPY_ROWS)
    def _zero(r):
        for j in range(C // L):
            rows0[r, pl.ds(j * L, L)] = zv

    for q in range(ROWS_PT // COPY_ROWS):
        pltpu.sync_copy(rows0.at[pl.ds(0, COPY_ROWS)],
                        acc.at[pl.ds(r0 + q * COPY_ROWS, COPY_ROWS)])

    pltpu.sync_copy(comb_hbm.at[t, 0], cbuf0)
    pltpu.sync_copy(w_hbm.at[t, 0], wbuf0)
    plsc.subcore_barrier()

    def start_refill(ch_, p):
        pltpu.async_copy(comb_hbm.at[t, ch_], cbuf[p], csem[p])
        pltpu.async_copy(w_hbm.at[t, ch_], wbuf[p], csem[p])

    def wait_refill(p):
        pltpu.make_async_copy(comb_hbm.at[t, 0], cbuf[p], csem[p]).wait()
        pltpu.make_async_copy(w_hbm.at[t, 0], wbuf[p], csem[p]).wait()

    def start_gather(idx_row, k):
        pltpu.async_copy(table.at[idx_row], rows[k], gsem[k])

    def wait_gather(k):
        pltpu.make_async_copy(table.at[cbuf0.at[0, 0]], rows[k],
                              gsem[k]).wait()

    def start_scatter(idx_row, k):
        pltpu.async_copy(rows[k], acc.at[idx_row], ssem[k], add=True)

    def wait_scatter(k):
        pltpu.make_async_copy(rows[k], acc.at[cbuf0.at[1, 0]],
                              ssem[k]).wait()

    _dnums = lax.GatherDimensionNumbers(
        offset_dims=(), collapsed_slice_dims=(0,), start_index_map=(0,))

    def scale(wb, i, k):
        rv = rows[k]

        @pl.loop(0, BLK // L)
        def _g(g):
            w16 = wb[i, pl.ds(g * L, L)]
            for e in range(L):
                ws = lax.gather(
                    w16, jnp.full((L, 1), e, jnp.int32), _dnums,
                    slice_sizes=(1,),
                    mode=lax.GatherScatterMode.PROMISE_IN_BOUNDS)
                r = g * L + e
                for j in range(C // L):
                    sl = pl.ds(j * L, L)
                    rv[r, sl] = rv[r, sl] * ws

    start_gather(cbuf0.at[0, 0], 0)

    @pl.loop(0, NCH, step=2)
    def _chunks(ch):
        for par in range(2):
            chh = ch + par
            cb = cbuf[par]
            cbn = cbuf[1 - par]
            wb = wbuf[par]
            more = chh + 1 < NCH
            last_pr = CHUNK // NBUF - 1

            @pl.loop(0, CHUNK // NBUF)
            def _pair(tr):
                for kk in range(NBUF):
                    i = tr * NBUF + kk
                    bb = chh * CHUNK + i

                    wait_gather(kk)
                    scale(wb, i, kk)

                    @pl.when(bb >= 1)
                    def _():
                        wait_scatter(1 - kk)

                    if kk == 0:
                        @pl.when(jnp.logical_and(tr == 0, more))
                        def _():
                            start_refill(chh + 1, 1 - par)

                        start_gather(cb.at[0, i + 1], 1 - kk)
                    else:
                        @pl.when(jnp.logical_and(tr == last_pr, more))
                        def _():
                            wait_refill(1 - par)

                        @pl.when(jnp.logical_not(tr == last_pr))
                        def _():
                            start_gather(cb.at[0, i + 1], 1 - kk)

                        @pl.when(jnp.logical_and(tr == last_pr, more))
                        def _():
                            start_gather(cbn.at[0, 0], 1 - kk)

                    start_scatter(cb.at[1, i], kk)

    wait_scatter((NB - 1) % NBUF)
    plsc.subcore_barrier()
    o0 = (1 - c) * NPAD + r0
    for q in range(ROWS_PT // COPY_ROWS):
        pltpu.sync_copy(acc.at[pl.ds(r0 + q * COPY_ROWS, COPY_ROWS)],
                        rows0.at[pl.ds(0, COPY_ROWS)])
        pltpu.sync_copy(rows0.at[pl.ds(0, COPY_ROWS)],
                        out.at[pl.ds(o0 + q * COPY_ROWS, COPY_ROWS)])


_mesh = plsc.VectorSubcoreMesh(core_axis_name="c", subcore_axis_name="s",
                               num_cores=NC, num_subcores=NS)

_layer = pl.kernel(
    _layer_body,
    out_type=jax.ShapeDtypeStruct((2 * NPAD, C), jnp.float32),
    mesh=_mesh,
    scratch_types=[
        pltpu.VMEM_SHARED((NPAD, C), jnp.float32),
        pltpu.VMEM((2, CHUNK, BLK), jnp.int32),
        pltpu.VMEM((2, CHUNK, BLK), jnp.int32),
        pltpu.VMEM((CHUNK, BLK), jnp.float32),
        pltpu.VMEM((CHUNK, BLK), jnp.float32),
        pltpu.VMEM((BLK, C), jnp.float32),
        pltpu.VMEM((BLK, C), jnp.float32),
        pltpu.SemaphoreType.DMA,
        pltpu.SemaphoreType.DMA,
        pltpu.SemaphoreType.DMA,
        pltpu.SemaphoreType.DMA,
        pltpu.SemaphoreType.DMA,
        pltpu.SemaphoreType.DMA,
    ],
)


def kernel(layers_num, user_emb, item_emb, inter_edge, inter_edge_w):
    src = inter_edge[0, :].astype(jnp.int32)
    dst = inter_edge[1, :].astype(jnp.int32)
    pad = EPAD - E
    zi = jnp.zeros((pad,), jnp.int32)
    src_p = jnp.concatenate([src, zi])
    dst_p = jnp.concatenate([dst, zi])
    w_p = jnp.concatenate([inter_edge_w.astype(jnp.float32),
                           jnp.zeros((pad,), jnp.float32)])
    shape5 = (2 * NS, NCH, CHUNK, BLK)
    gidx = jnp.concatenate([dst_p, src_p + NPAD]).reshape(shape5)
    sidx = jnp.concatenate([src_p, dst_p]).reshape(shape5)
    comb = jnp.stack([gidx, sidx], axis=2)
    warr = jnp.concatenate([w_p, w_p]).reshape(shape5)

    zrow = jnp.zeros((NPAD - N, C), jnp.float32)
    table = jnp.concatenate([item_emb, zrow, user_emb, zrow], axis=0)
    total = table
    for _ in range(3):
        table = _layer(table, comb, warr)
        total = total + table

    denom = (jnp.asarray(layers_num) + 1).astype(jnp.float32)
    item_out = total[:N] / denom
    user_out = total[NPAD:NPAD + N] / denom
    return (item_out, user_out)


# speedup vs baseline: 1.8134x; 1.8134x over previous
import jax
import jax.numpy as jnp
from jax import lax
from jax.experimental import pallas as pl
from jax.experimental.pallas import tpu as pltpu
from jax.experimental.pallas import tpu_sc as plsc

N = 10000
NPAD = 10240
C = 128
E = 320000
NC = 2
NS = 16
L = 16
BLK = 96
NBUF = 3
CHUNK = 15
NB = 210
NCH = NB // CHUNK
EPT = NB * BLK
EPAD = EPT * NS
ROWS_PT = NPAD // NS
COPY_ROWS = 64


def _layer_body(table, comb_hbm, w_hbm, out, acc, cbuf0, cbuf1,
                wbuf0, wbuf1, rows0, rows1, rows2,
                gs0, gs1, gs2, ss0, ss1, ss2, cs0, cs1):
    c = lax.axis_index("c")
    s = lax.axis_index("s")
    rows = (rows0, rows1, rows2)
    gsem = (gs0, gs1, gs2)
    ssem = (ss0, ss1, ss2)
    cbuf = (cbuf0, cbuf1)
    wbuf = (wbuf0, wbuf1)
    csem = (cs0, cs1)
    t = c * NS + s
    r0 = s * ROWS_PT

    zv = jnp.zeros((L,), jnp.float32)

    @pl.loop(0, COPY_ROWS)
    def _zero(r):
        for j in range(C // L):
            rows0[r, pl.ds(j * L, L)] = zv

    for q in range(ROWS_PT // COPY_ROWS):
        pltpu.sync_copy(rows0.at[pl.ds(0, COPY_ROWS)],
                        acc.at[pl.ds(r0 + q * COPY_ROWS, COPY_ROWS)])

    pltpu.sync_copy(comb_hbm.at[t, 0], cbuf0)
    pltpu.sync_copy(w_hbm.at[t, 0], wbuf0)
    plsc.subcore_barrier()

    def start_refill(ch_, p):
        pltpu.async_copy(comb_hbm.at[t, ch_], cbuf[p], csem[p])
        pltpu.async_copy(w_hbm.at[t, ch_], wbuf[p], csem[p])

    def wait_refill(p):
        pltpu.make_async_copy(comb_hbm.at[t, 0], cbuf[p], csem[p]).wait()
        pltpu.make_async_copy(w_hbm.at[t, 0], wbuf[p], csem[p]).wait()

    def start_gather(idx_row, k):
        pltpu.async_copy(table.at[idx_row], rows[k], gsem[k])

    def wait_gather(k):
        pltpu.make_async_copy(table.at[cbuf0.at[0, 0]], rows[k],
                              gsem[k]).wait()

    def start_scatter(idx_row, k):
        pltpu.async_copy(rows[k], acc.at[idx_row], ssem[k], add=True)

    def wait_scatter(k):
        pltpu.make_async_copy(rows[k], acc.at[cbuf0.at[1, 0]],
                              ssem[k]).wait()

    _dnums = lax.GatherDimensionNumbers(
        offset_dims=(), collapsed_slice_dims=(0,), start_index_map=(0,))

    def scale(wb, i, k):
        rv = rows[k]

        @pl.loop(0, BLK // L)
        def _g(g):
            w16 = wb[i, pl.ds(g * L, L)]
            for e in range(L):
                ws = lax.gather(
                    w16, jnp.full((L, 1), e, jnp.int32), _dnums,
                    slice_sizes=(1,),
                    mode=lax.GatherScatterMode.PROMISE_IN_BOUNDS)
                r = g * L + e
                for j in range(C // L):
                    sl = pl.ds(j * L, L)
                    rv[r, sl] = rv[r, sl] * ws

    start_gather(cbuf0.at[0, 0], 0)
    start_gather(cbuf0.at[0, 1], 1)

    @pl.loop(0, NCH, step=2)
    def _chunks(ch):
        for par in range(2):
            chh = ch + par
            cb = cbuf[par]
            cbn = cbuf[1 - par]
            wb = wbuf[par]
            more = chh + 1 < NCH

            @pl.loop(0, CHUNK // NBUF)
            def _triple(tr):
                for kk in range(NBUF):
                    i = tr * NBUF + kk
                    bb = chh * CHUNK + i
                    kp = (kk + 2) % NBUF

                    wait_gather(kk)
                    scale(wb, i, kk)

                    @pl.when(bb >= 1)
                    def _():
                        wait_scatter(kp)

                    if kk == 0:
                        @pl.when(jnp.logical_and(tr == 0, more))
                        def _():
                            start_refill(chh + 1, 1 - par)

                        start_gather(cb.at[0, i + 2], kp)
                    else:
                        last_tr = tr == CHUNK // NBUF - 1
                        if kk == 1:
                            @pl.when(jnp.logical_and(last_tr, more))
                            def _():
                                wait_refill(1 - par)

                        @pl.when(jnp.logical_not(last_tr))
                        def _():
                            start_gather(cb.at[0, i + 2], kp)

                        @pl.when(jnp.logical_and(last_tr, more))
                        def _():
                            start_gather(cbn.at[0, i - (CHUNK - 2)], kp)

                    start_scatter(cb.at[1, i], kk)

    wait_scatter((NB - 1) % NBUF)
    plsc.subcore_barrier()
    o0 = (1 - c) * NPAD + r0
    for q in range(ROWS_PT // COPY_ROWS):
        pltpu.sync_copy(acc.at[pl.ds(r0 + q * COPY_ROWS, COPY_ROWS)],
                        rows0.at[pl.ds(0, COPY_ROWS)])
        pltpu.sync_copy(rows0.at[pl.ds(0, COPY_ROWS)],
                        out.at[pl.ds(o0 + q * COPY_ROWS, COPY_ROWS)])


_mesh = plsc.VectorSubcoreMesh(core_axis_name="c", subcore_axis_name="s",
                               num_cores=NC, num_subcores=NS)

_layer = pl.kernel(
    _layer_body,
    out_type=jax.ShapeDtypeStruct((2 * NPAD, C), jnp.float32),
    mesh=_mesh,
    scratch_types=[
        pltpu.VMEM_SHARED((NPAD, C), jnp.float32),
        pltpu.VMEM((2, CHUNK, BLK), jnp.int32),
        pltpu.VMEM((2, CHUNK, BLK), jnp.int32),
        pltpu.VMEM((CHUNK, BLK), jnp.float32),
        pltpu.VMEM((CHUNK, BLK), jnp.float32),
        pltpu.VMEM((BLK, C), jnp.float32),
        pltpu.VMEM((BLK, C), jnp.float32),
        pltpu.VMEM((BLK, C), jnp.float32),
        pltpu.SemaphoreType.DMA,
        pltpu.SemaphoreType.DMA,
        pltpu.SemaphoreType.DMA,
        pltpu.SemaphoreType.DMA,
        pltpu.SemaphoreType.DMA,
        pltpu.SemaphoreType.DMA,
        pltpu.SemaphoreType.DMA,
        pltpu.SemaphoreType.DMA,
    ],
)


def kernel(layers_num, user_emb, item_emb, inter_edge, inter_edge_w):
    src = inter_edge[0, :].astype(jnp.int32)
    dst = inter_edge[1, :].astype(jnp.int32)
    pad = EPAD - E
    zi = jnp.zeros((pad,), jnp.int32)
    src_p = jnp.concatenate([src, zi])
    dst_p = jnp.concatenate([dst, zi])
    w_p = jnp.concatenate([inter_edge_w.astype(jnp.float32),
                           jnp.zeros((pad,), jnp.float32)])
    shape5 = (2 * NS, NCH, CHUNK, BLK)
    gidx = jnp.concatenate([dst_p, src_p + NPAD]).reshape(shape5)
    sidx = jnp.concatenate([src_p, dst_p]).reshape(shape5)
    comb = jnp.stack([gidx, sidx], axis=2)
    warr = jnp.concatenate([w_p, w_p]).reshape(shape5)

    zrow = jnp.zeros((NPAD - N, C), jnp.float32)
    table = jnp.concatenate([item_emb, zrow, user_emb, zrow], axis=0)
    total = table
    for _ in range(3):
        table = _layer(table, comb, warr)
        total = total + table

    denom = (jnp.asarray(layers_num) + 1).astype(jnp.float32)
    item_out = total[:N] / denom
    user_out = total[NPAD:NPAD + N] / denom
    return (item_out, user_out)
